# Initial kernel scaffold; baseline (speedup 1.0000x reference)
import jax
import jax.numpy as jnp
from jax.experimental import pallas as pl


def kernel(u, i, j, adj_rows, adj_cols, adj_vals, user_emb, item_emb, W1_0, b1_0, W2_0, b2_0, W1_1, b1_1, W2_1, b2_1):
    raise NotImplementedError("write your pallas kernel here")


# speedup vs baseline: 7.0924x; 7.0924x over previous
import functools

import jax
import jax.numpy as jnp
from jax import lax
from jax.experimental import pallas as pl
from jax.experimental.pallas import tpu as pltpu
from jax.experimental.pallas import tpu_sc as plsc

N_USERS = 50000
N_ITEMS = 50000
N = N_USERS + N_ITEMS
E = 1600000
D = 32
B = 4096

NC = 2
NS = 16
N2 = N // NC
ACC_ROWS = 50048
ZROWS = ACC_ROWS // NS
WROWS8 = 3128
WROWS_LAST = N2 - (NS - 1) * WROWS8
K = 512
GD = K // 128
CHUNKS = 196
EP = K * CHUNKS
E_PAD = EP * NS

@functools.cache
def _mesh():
    return plsc.VectorSubcoreMesh(core_axis_name="c", subcore_axis_name="s",
                                  num_cores=NC, num_subcores=NS)


def _seg_body(ego_hbm, cols_hbm, rows_hbm, vals_hbm, msg_hbm,
              acc, cols_v, rows_v, vals_v, lidx, gath, sem):
    cid = lax.axis_index("c")
    sid = lax.axis_index("s")
    base = cid * N2

    @pl.loop(0, K)
    def _zero(e):
        z = jnp.zeros((16,), jnp.float32)
        gath[e, pl.ds(0, 16)] = z
        gath[e, pl.ds(16, 16)] = z

    z0 = sid * ZROWS
    for off in range(0, ZROWS, K):
        size = min(K, ZROWS - off)
        pltpu.sync_copy(gath.at[pl.ds(0, size), :],
                        acc.at[pl.ds(z0 + off, size), :])
    plsc.subcore_barrier()

    @pl.loop(0, CHUNKS)
    def _chunk(ci):
        e0 = sid * EP + ci * K
        pltpu.sync_copy(cols_hbm.at[pl.ds(e0, K)], cols_v)
        pltpu.sync_copy(rows_hbm.at[pl.ds(e0, K)], rows_v)
        pltpu.sync_copy(vals_hbm.at[pl.ds(e0, K)], vals_v)

        @pl.loop(0, GD)
        def _fire_g(d):
            pltpu.async_copy(ego_hbm.at[cols_v.at[pl.ds(d * 128, 128)]],
                             gath.at[pl.ds(d * 128, 128), :], sem)

        @pl.loop(0, K // 16)
        def _lidx(i):
            r = rows_v[pl.ds(i * 16, 16)]
            loc = r - base
            ok = (loc >= 0) & (loc < N2)
            idx = jnp.where(ok, loc, N2)
            lidx[i // 8, pl.ds((i % 8) * 16, 16)] = idx

        @pl.loop(0, GD)
        def _drain_g(d):
            pltpu.make_async_copy(
                ego_hbm.at[cols_v.at[pl.ds(d * 128, 128)]],
                gath.at[pl.ds(d * 128, 128), :], sem).wait()

        @pl.loop(0, K // 16)
        def _scale(i):
            vv = vals_v[pl.ds(i * 16, 16)]
            for r in range(16):
                e = i * 16 + r
                v = vv[r]
                gath[e, pl.ds(0, 16)] = gath[e, pl.ds(0, 16)] * v
                gath[e, pl.ds(16, 16)] = gath[e, pl.ds(16, 16)] * v

        @pl.loop(0, GD)
        def _fire_s(d):
            pltpu.async_copy(gath.at[pl.ds(d * 128, 128), :],
                             acc.at[lidx.at[d]], sem, add=True)

        @pl.loop(0, GD)
        def _drain_s(d):
            pltpu.make_async_copy(gath.at[pl.ds(d * 128, 128), :],
                                  acc.at[lidx.at[d]], sem).wait()

    plsc.subcore_barrier()

    w0 = sid * WROWS8

    @pl.when(sid < NS - 1)
    def _wb_full():
        pltpu.sync_copy(acc.at[pl.ds(w0, WROWS8), :],
                        msg_hbm.at[pl.ds(base + w0, WROWS8), :])

    @pl.when(sid == NS - 1)
    def _wb_last():
        pltpu.sync_copy(acc.at[pl.ds(w0, WROWS_LAST), :],
                        msg_hbm.at[pl.ds(base + w0, WROWS_LAST), :])


@functools.cache
def _seg():
    return pl.kernel(
        _seg_body,
        out_type=jax.ShapeDtypeStruct((N, D), jnp.float32),
        mesh=_mesh(),
        scratch_types=[
            pltpu.VMEM_SHARED((ACC_ROWS, D), jnp.float32),
            pltpu.VMEM((K,), jnp.int32),
            pltpu.VMEM((K,), jnp.int32),
            pltpu.VMEM((K,), jnp.float32),
            pltpu.VMEM((GD, 128), jnp.int32),
            pltpu.VMEM((K, D), jnp.float32),
            pltpu.SemaphoreType.DMA,
        ],
        compiler_params=pltpu.CompilerParams(use_tc_tiling_on_sc=False),
    )


RB = 2000


def _dense_body(ego_ref, msg_ref, w1_ref, b1_ref, w2_ref, b2_ref,
                out_ref, nrm_ref):
    e = ego_ref[...]
    m = msg_ref[...]
    t = jnp.dot(m, w1_ref[...], preferred_element_type=jnp.float32)
    t = t + jnp.dot(e * m, w2_ref[...], preferred_element_type=jnp.float32)
    t = t + b1_ref[...] + b2_ref[...]
    act = jnp.where(t >= 0, t, 0.01 * t)
    out_ref[...] = act
    nn = jnp.sqrt(jnp.sum(act * act, axis=1, keepdims=True))
    nrm_ref[...] = act / jnp.maximum(nn, 1e-12)


def _dense(ego, msg, W1, b1, W2, b2):
    bs_rows = pl.BlockSpec((RB, D), lambda i: (i, 0))
    bs_w = pl.BlockSpec((D, D), lambda i: (0, 0))
    bs_b = pl.BlockSpec((1, D), lambda i: (0, 0))
    return pl.pallas_call(
        _dense_body,
        grid=(N // RB,),
        in_specs=[bs_rows, bs_rows, bs_w, bs_b, bs_w, bs_b],
        out_specs=[bs_rows, bs_rows],
        out_shape=[jax.ShapeDtypeStruct((N, D), jnp.float32)] * 2,
    )(ego, msg, W1, b1.reshape(1, D), W2, b2.reshape(1, D))


BW = B // (NC * NS)


def _gat_body(e0_hbm, n1_hbm, n2_hbm, u_hbm, i_hbm, j_hbm,
              uo_hbm, po_hbm, no_hbm, idx_v, gbuf, obuf, sem):
    cid = lax.axis_index("c")
    sid = lax.axis_index("s")
    wid = sid * NC + cid
    r0 = wid * BW

    for idx_hbm, out_hbm, ibase in ((u_hbm, uo_hbm, 0),
                                    (i_hbm, po_hbm, N_USERS),
                                    (j_hbm, no_hbm, N_USERS)):
        pltpu.sync_copy(idx_hbm.at[pl.ds(r0, BW)], idx_v)
        if ibase:
            @pl.loop(0, BW // 16)
            def _shift(k):
                idx_v[pl.ds(k * 16, 16)] = idx_v[pl.ds(k * 16, 16)] + ibase

        for ti, tab in enumerate((e0_hbm, n1_hbm, n2_hbm)):
            pltpu.async_copy(tab.at[idx_v], gbuf, sem).wait()

            @pl.loop(0, BW)
            def _pack(k):
                obuf[k, pl.ds(ti * 32, 16)] = gbuf[k, pl.ds(0, 16)]
                obuf[k, pl.ds(ti * 32 + 16, 16)] = gbuf[k, pl.ds(16, 16)]

        pltpu.sync_copy(obuf, out_hbm.at[pl.ds(r0, BW), :])


@functools.cache
def _gather3():
    return pl.kernel(
        _gat_body,
        out_type=(jax.ShapeDtypeStruct((B, 3 * D), jnp.float32),) * 3,
        mesh=_mesh(),
        scratch_types=[
            pltpu.VMEM((BW,), jnp.int32),
            pltpu.VMEM((BW, D), jnp.float32),
            pltpu.VMEM((BW, 3 * D), jnp.float32),
            pltpu.SemaphoreType.DMA,
        ],
        compiler_params=pltpu.CompilerParams(use_tc_tiling_on_sc=False),
    )


def kernel(u, i, j, adj_rows, adj_cols, adj_vals, user_emb, item_emb,
           W1_0, b1_0, W2_0, b2_0, W1_1, b1_1, W2_1, b2_1):
    ego0 = jnp.concatenate([user_emb, item_emb], axis=0)
    pad = E_PAD - E
    cols_p = jnp.concatenate([adj_cols, jnp.zeros((pad,), jnp.int32)])
    rows_p = jnp.concatenate([adj_rows, jnp.full((pad,), N, jnp.int32)])
    vals_p = jnp.concatenate([adj_vals, jnp.zeros((pad,), jnp.float32)])

    ego = ego0
    normed = []
    for (W1, b1, W2, b2) in ((W1_0, b1_0, W2_0, b2_0),
                             (W1_1, b1_1, W2_1, b2_1)):
        msg = _seg()(ego, cols_p, rows_p, vals_p)
        ego, nrm = _dense(ego, msg, W1, b1, W2, b2)
        normed.append(nrm)

    return _gather3()(ego0, normed[0], normed[1], u, i, j)
